# baseline (speedup 1.0000x reference)
import jax
import jax.numpy as jnp
from jax import lax
from jax.experimental import pallas as pl
from jax.experimental.pallas import tpu as pltpu
from jax.experimental.pallas import tpu_sc as plsc

_L, _B, _D = 200, 4096, 64
_BW = 128
_CPB = 8
_NBLK = _L // _CPB
_TS = 129


def _transpose_chunk(gbuf, tbuf):
    lane = lax.iota(jnp.int32, 16)
    cvecs = [c0 * 16 + lane for c0 in range(_D // 16)]
    zero = jnp.zeros((16,), jnp.int32)

    def body(bi, carry):
        work = []
        for db in range(2):
            b = 2 * bi + db
            bidx = zero + b
            for c0 in range(_D // 16):
                v = gbuf[b, pl.ds(c0 * 16, 16)]
                work.append((c0, bidx, v))
        for c0, bidx, v in work:
            plsc.store_scatter(tbuf, [cvecs[c0], bidx], v)
        return carry

    lax.fori_loop(0, _BW // 2, body, 0)


def _emb_body(table_hbm, idx_hbm, out_hbm,
              idx_v, ga, gb, gc, gd, ta, tb,
              isem, gsa, gsb, gsc, gsd, ssa, ssb):
    wid = lax.axis_index("s") * 2 + lax.axis_index("c")
    b0 = wid * _BW

    def istart(blk):
        pltpu.async_copy(
            idx_hbm.at[pl.ds(blk * _CPB, _CPB), pl.ds(b0, _BW)], idx_v, isem)

    def iwait():
        pltpu.make_async_copy(
            idx_hbm.at[pl.ds(0, _CPB), pl.ds(b0, _BW)], idx_v, isem).wait()

    def gstart(u, buf, sem):
        pltpu.async_copy(table_hbm.at[idx_v.at[u]], buf, sem)

    def gwait(u, buf, sem):
        pltpu.make_async_copy(table_hbm.at[idx_v.at[u]], buf, sem).wait()

    def sstart(c, buf, sem):
        pltpu.async_copy(
            buf.at[:, pl.ds(0, _BW)],
            out_hbm.at[c, :, pl.ds(b0, _BW)], sem)

    def swait(c, buf, sem):
        pltpu.make_async_copy(
            buf.at[:, pl.ds(0, _BW)],
            out_hbm.at[c, :, pl.ds(b0, _BW)], sem
        ).wait()

    istart(0)

    def block(blk, carry):
        c_base = blk * _CPB
        iwait()
        gbufs = (ga, gb, gc, gd)
        gsems = (gsa, gsb, gsc, gsd)
        tbufs = (ta, tb)
        ssems = (ssa, ssb)
        for u in range(3):
            gstart(u, gbufs[u % 4], gsems[u % 4])
        for u in range(_CPB):
            g, gs = gbufs[u % 4], gsems[u % 4]
            t, ss = tbufs[u % 2], ssems[u % 2]
            if u + 3 < _CPB:
                gstart(u + 3, gbufs[(u + 3) % 4], gsems[(u + 3) % 4])
            gwait(u, g, gs)
            if u >= 2:
                swait(c_base + u - 2, t, ss)
            else:
                @pl.when(blk > 0)
                def _():
                    swait(c_base + u - 2, t, ss)
            if u == _CPB - 1:
                @pl.when(blk + 1 < _NBLK)
                def _():
                    istart(blk + 1)
            _transpose_chunk(g, t)
            sstart(c_base + u, t, ss)
        return carry

    lax.fori_loop(0, _NBLK, block, 0)
    swait(_NBLK * _CPB - 2, ta, ssa)
    swait(_NBLK * _CPB - 1, tb, ssb)


def kernel(indices, labels, table):
    idx = indices.astype(jnp.int32)
    table128 = jnp.pad(table, ((0, 0), (0, 128 - _D)))
    mesh = plsc.VectorSubcoreMesh(core_axis_name="c", subcore_axis_name="s")
    out = pl.kernel(
        _emb_body,
        mesh=mesh,
        compiler_params=pltpu.CompilerParams(
            use_tc_tiling_on_sc=True, needs_layout_passes=False),
        out_type=jax.ShapeDtypeStruct((_L, _D, _B), jnp.float32),
        scratch_types=[
            pltpu.VMEM((_CPB, _BW), jnp.int32),
            pltpu.VMEM((_BW, 128), jnp.float32),
            pltpu.VMEM((_BW, 128), jnp.float32),
            pltpu.VMEM((_BW, 128), jnp.float32),
            pltpu.VMEM((_BW, 128), jnp.float32),
            pltpu.VMEM((_D, _TS), jnp.float32),
            pltpu.VMEM((_D, _TS), jnp.float32),
            pltpu.SemaphoreType.DMA,
            pltpu.SemaphoreType.DMA,
            pltpu.SemaphoreType.DMA,
            pltpu.SemaphoreType.DMA,
            pltpu.SemaphoreType.DMA,
            pltpu.SemaphoreType.DMA,
            pltpu.SemaphoreType.DMA,
        ],
    )(table128, idx)
    return (jnp.transpose(out, (0, 2, 1)), labels)


# speedup vs baseline: 1.6340x; 1.6340x over previous
import jax
import jax.numpy as jnp
from jax import lax
from jax.experimental import pallas as pl
from jax.experimental.pallas import tpu as pltpu
from jax.experimental.pallas import tpu_sc as plsc

_L, _B, _D = 200, 4096, 64
_BW = 128
_CL = 2
_NCH = _L // _CL
_NG = _NCH // 4


def _emb_body(table_hbm, idx_hbm, out_hbm, idx_v, g0, g1, g2, g3,
              gs0, gs1, gs2, gs3, ss0, ss1, ss2, ss3):
    wid = lax.axis_index("s") * 2 + lax.axis_index("c")
    b0 = wid * _BW
    pltpu.sync_copy(idx_hbm.at[:, pl.ds(b0, _BW)], idx_v)

    gbufs = (g0, g1, g2, g3)
    gsems = (gs0, gs1, gs2, gs3)
    ssems = (ss0, ss1, ss2, ss3)

    def gstart(c, q):
        for j in range(_CL):
            pltpu.async_copy(
                table_hbm.at[idx_v.at[c * _CL + j]], gbufs[q].at[j], gsems[q])

    def gwait(c, q):
        for j in range(_CL):
            pltpu.make_async_copy(
                table_hbm.at[idx_v.at[c * _CL + j]], gbufs[q].at[j], gsems[q]
            ).wait()

    def sstart(c, q):
        pltpu.async_copy(
            gbufs[q],
            out_hbm.at[pl.ds(c * _CL, _CL), pl.ds(b0, _BW), :], ssems[q])

    def swait(c, q):
        pltpu.make_async_copy(
            gbufs[q],
            out_hbm.at[pl.ds(c * _CL, _CL), pl.ds(b0, _BW), :], ssems[q]
        ).wait()

    gstart(0, 0)
    gstart(1, 1)

    def group(g, carry):
        cb = 4 * g
        for q in range(4):
            c = cb + q
            nq = (q + 2) % 4
            if q >= 2:
                swait(c - 2, nq)
            else:
                @pl.when(g > 0)
                def _():
                    swait(c - 2, nq)

            @pl.when(c + 2 < _NCH)
            def _():
                gstart(c + 2, nq)

            gwait(c, q)
            sstart(c, q)
        return carry

    lax.fori_loop(0, _NG, group, 0)
    swait(_NCH - 2, 2)
    swait(_NCH - 1, 3)


def kernel(indices, labels, table):
    idx = indices.astype(jnp.int32)
    mesh = plsc.VectorSubcoreMesh(core_axis_name="c", subcore_axis_name="s")
    out = pl.kernel(
        _emb_body,
        mesh=mesh,
        compiler_params=pltpu.CompilerParams(use_tc_tiling_on_sc=False),
        out_type=jax.ShapeDtypeStruct((_L, _B, _D), jnp.float32),
        scratch_types=[
            pltpu.VMEM((_L, _BW), jnp.int32),
            pltpu.VMEM((_CL, _BW, _D), jnp.float32),
            pltpu.VMEM((_CL, _BW, _D), jnp.float32),
            pltpu.VMEM((_CL, _BW, _D), jnp.float32),
            pltpu.VMEM((_CL, _BW, _D), jnp.float32),
            pltpu.SemaphoreType.DMA,
            pltpu.SemaphoreType.DMA,
            pltpu.SemaphoreType.DMA,
            pltpu.SemaphoreType.DMA,
            pltpu.SemaphoreType.DMA,
            pltpu.SemaphoreType.DMA,
            pltpu.SemaphoreType.DMA,
            pltpu.SemaphoreType.DMA,
        ],
    )(table, idx)
    return (out, labels)


# speedup vs baseline: 2.1634x; 1.3240x over previous
import jax
import jax.numpy as jnp
from jax import lax
from jax.experimental import pallas as pl
from jax.experimental.pallas import tpu as pltpu
from jax.experimental.pallas import tpu_sc as plsc

_L, _B, _D = 200, 4096, 64
_BW = 128
_NCH = _L
_NG = _NCH // 4


def _emb_body(table_hbm, idx_hbm, out_hbm, idx_v, g0, g1, g2, g3,
              gs0, gs1, gs2, gs3, ss0, ss1, ss2, ss3):
    wid = lax.axis_index("s") * 2 + lax.axis_index("c")
    b0 = wid * _BW
    pltpu.sync_copy(idx_hbm.at[:, pl.ds(b0, _BW)], idx_v)

    gbufs = (g0, g1, g2, g3)
    gsems = (gs0, gs1, gs2, gs3)
    ssems = (ss0, ss1, ss2, ss3)

    def gstart(c, q):
        pltpu.async_copy(table_hbm.at[idx_v.at[c]], gbufs[q], gsems[q])

    def gwait(c, q):
        pltpu.make_async_copy(
            table_hbm.at[idx_v.at[c]], gbufs[q], gsems[q]).wait()

    def sstart(c, q):
        pltpu.async_copy(
            gbufs[q], out_hbm.at[c, pl.ds(b0, _BW), :], ssems[q])

    def swait(c, q):
        pltpu.make_async_copy(
            gbufs[q], out_hbm.at[c, pl.ds(b0, _BW), :], ssems[q]).wait()

    gstart(0, 0)
    gstart(1, 1)

    def group(g, carry):
        cb = 4 * g
        for q in range(4):
            c = cb + q
            nq = (q + 2) % 4
            if q >= 2:
                swait(c - 2, nq)
            else:
                @pl.when(g > 0)
                def _():
                    swait(c - 2, nq)

            @pl.when(c + 2 < _NCH)
            def _():
                gstart(c + 2, nq)

            gwait(c, q)
            sstart(c, q)
        return carry

    lax.fori_loop(0, _NG, group, 0)
    swait(_NCH - 2, 2)
    swait(_NCH - 1, 3)


def kernel(indices, labels, table):
    idx = indices.astype(jnp.int32)
    table128 = jnp.pad(table, ((0, 0), (0, 128 - _D)))
    mesh = plsc.VectorSubcoreMesh(core_axis_name="c", subcore_axis_name="s")
    out = pl.kernel(
        _emb_body,
        mesh=mesh,
        compiler_params=pltpu.CompilerParams(
            use_tc_tiling_on_sc=True, needs_layout_passes=False),
        out_type=jax.ShapeDtypeStruct((_L, _B, 128), jnp.float32),
        scratch_types=[
            pltpu.VMEM((_L, _BW), jnp.int32),
            pltpu.VMEM((_BW, 128), jnp.float32),
            pltpu.VMEM((_BW, 128), jnp.float32),
            pltpu.VMEM((_BW, 128), jnp.float32),
            pltpu.VMEM((_BW, 128), jnp.float32),
            pltpu.SemaphoreType.DMA,
            pltpu.SemaphoreType.DMA,
            pltpu.SemaphoreType.DMA,
            pltpu.SemaphoreType.DMA,
            pltpu.SemaphoreType.DMA,
            pltpu.SemaphoreType.DMA,
            pltpu.SemaphoreType.DMA,
            pltpu.SemaphoreType.DMA,
        ],
    )(table128, idx)
    return (out[:, :, :_D], labels)
